# initial kernel scaffold (unmeasured)
import jax
import jax.numpy as jnp
from jax import lax
from jax.experimental import pallas as pl
from jax.experimental.pallas import tpu as pltpu

N_DEV = 4


def kernel(x, w_mat, scale_x, scale_w):
    m_per, k = x.shape
    _, n_per = w_mat.shape

    def body(x_ref, w_ref, sx_ref, sw_ref, out_ref, comm_ref, send_sems, recv_sems):
        my_pos = lax.axis_index("i")
        left = lax.rem(my_pos + (N_DEV - 1), N_DEV)
        right = lax.rem(my_pos + 1, N_DEV)

        barrier_sem = pltpu.get_barrier_semaphore()
        for nbr in (left, right):
            pl.semaphore_signal(
                barrier_sem, inc=1,
                device_id=(nbr,), device_id_type=pl.DeviceIdType.MESH,
            )
        pl.semaphore_wait(barrier_sem, 2)

        scale = sx_ref[0] * sw_ref[0]

        for h in range(N_DEV - 1):
            src = x_ref if h == 0 else comm_ref.at[h - 1]
            rdma = pltpu.make_async_remote_copy(
                src_ref=src,
                dst_ref=comm_ref.at[h],
                send_sem=send_sems.at[h],
                recv_sem=recv_sems.at[h],
                device_id=(right,),
                device_id_type=pl.DeviceIdType.MESH,
            )
            rdma.start()
            held = x_ref[...] if h == 0 else comm_ref[h - 1]
            origin = lax.rem(my_pos + (N_DEV - h), N_DEV) if h else my_pos
            out_ref[pl.ds(origin * m_per, m_per), :] = (
                jnp.dot(held, w_ref[...], preferred_element_type=jnp.float32)
                * scale
            )
            rdma.wait()

        origin = lax.rem(my_pos + 1, N_DEV)
        out_ref[pl.ds(origin * m_per, m_per), :] = (
            jnp.dot(comm_ref[N_DEV - 2], w_ref[...],
                    preferred_element_type=jnp.float32)
            * scale
        )

    return pl.pallas_call(
        body,
        out_shape=jax.ShapeDtypeStruct((N_DEV * m_per, n_per), jnp.float32),
        in_specs=[
            pl.BlockSpec(memory_space=pltpu.VMEM),
            pl.BlockSpec(memory_space=pltpu.VMEM),
            pl.BlockSpec(memory_space=pltpu.SMEM),
            pl.BlockSpec(memory_space=pltpu.SMEM),
        ],
        out_specs=pl.BlockSpec(memory_space=pltpu.VMEM),
        scratch_shapes=[
            pltpu.VMEM((N_DEV - 1, m_per, k), x.dtype),
            pltpu.SemaphoreType.DMA((N_DEV - 1,)),
            pltpu.SemaphoreType.DMA((N_DEV - 1,)),
        ],
        compiler_params=pltpu.CompilerParams(
            collective_id=0,
            vmem_limit_bytes=100 * 1024 * 1024,
        ),
    )(x, w_mat, scale_x, scale_w)


# baseline (device time: 213856 ns/iter reference)
import jax
import jax.numpy as jnp
from jax import lax
from jax.experimental import pallas as pl
from jax.experimental.pallas import tpu as pltpu

N_DEV = 4


def kernel(x, w_mat, scale_x, scale_w):
    m_per, k = x.shape
    _, n_per = w_mat.shape

    x = x.astype(jnp.float8_e5m2)
    w_mat = w_mat.astype(jnp.float8_e5m2)

    def body(x_ref, w_ref, sx_ref, sw_ref, out_ref, comm_ref, send_sems, recv_sems):
        my_pos = lax.axis_index("i")
        left = lax.rem(my_pos + (N_DEV - 1), N_DEV)
        right = lax.rem(my_pos + 1, N_DEV)

        barrier_sem = pltpu.get_barrier_semaphore()
        for nbr in (left, right):
            pl.semaphore_signal(
                barrier_sem, inc=1,
                device_id=(nbr,), device_id_type=pl.DeviceIdType.MESH,
            )
        pl.semaphore_wait(barrier_sem, 2)

        scale = sx_ref[0] * sw_ref[0]

        for h in range(N_DEV - 1):
            src = x_ref if h == 0 else comm_ref.at[h - 1]
            rdma = pltpu.make_async_remote_copy(
                src_ref=src,
                dst_ref=comm_ref.at[h],
                send_sem=send_sems.at[h],
                recv_sem=recv_sems.at[h],
                device_id=(right,),
                device_id_type=pl.DeviceIdType.MESH,
            )
            rdma.start()
            held = x_ref[...] if h == 0 else comm_ref[h - 1]
            origin = lax.rem(my_pos + (N_DEV - h), N_DEV) if h else my_pos
            out_ref[pl.ds(origin * m_per, m_per), :] = (
                jnp.dot(held, w_ref[...], preferred_element_type=jnp.float32)
                * scale
            )
            rdma.wait()

        origin = lax.rem(my_pos + 1, N_DEV)
        out_ref[pl.ds(origin * m_per, m_per), :] = (
            jnp.dot(comm_ref[N_DEV - 2], w_ref[...],
                    preferred_element_type=jnp.float32)
            * scale
        )

    return pl.pallas_call(
        body,
        out_shape=jax.ShapeDtypeStruct((N_DEV * m_per, n_per), jnp.float32),
        in_specs=[
            pl.BlockSpec(memory_space=pltpu.VMEM),
            pl.BlockSpec(memory_space=pltpu.VMEM),
            pl.BlockSpec(memory_space=pltpu.SMEM),
            pl.BlockSpec(memory_space=pltpu.SMEM),
        ],
        out_specs=pl.BlockSpec(memory_space=pltpu.VMEM),
        scratch_shapes=[
            pltpu.VMEM((N_DEV - 1, m_per, k), x.dtype),
            pltpu.SemaphoreType.DMA((N_DEV - 1,)),
            pltpu.SemaphoreType.DMA((N_DEV - 1,)),
        ],
        compiler_params=pltpu.CompilerParams(
            collective_id=0,
            vmem_limit_bytes=100 * 1024 * 1024,
        ),
    )(x, w_mat, scale_x, scale_w)


# device time: 144328 ns/iter; 1.4817x vs baseline; 1.4817x over previous
import jax
import jax.numpy as jnp
from jax import lax
from jax.experimental import pallas as pl
from jax.experimental.pallas import tpu as pltpu

N_DEV = 4


def kernel(x, w_mat, scale_x, scale_w):
    m_per, k = x.shape
    _, n_per = w_mat.shape
    half = m_per // 2

    x = x.astype(jnp.float8_e5m2)
    w_mat = w_mat.astype(jnp.float8_e5m2)

    def body(x_ref, w_ref, sx_ref, sw_ref, out_ref, comm_ref, send_sems, recv_sems):
        my = lax.axis_index("i")
        left = lax.rem(my + (N_DEV - 1), N_DEV)
        right = lax.rem(my + 1, N_DEV)

        barrier_sem = pltpu.get_barrier_semaphore()
        for nbr in (left, right):
            pl.semaphore_signal(
                barrier_sem, inc=1,
                device_id=(nbr,), device_id_type=pl.DeviceIdType.MESH,
            )
        pl.semaphore_wait(barrier_sem, 2)

        scale = sx_ref[0] * sw_ref[0]

        def gemm(chunk, origin):
            out_ref[pl.ds(origin * m_per, m_per), :] = (
                jnp.dot(chunk, w_ref[...], preferred_element_type=jnp.float32)
                * scale
            )

        r1l = pltpu.make_async_remote_copy(
            src_ref=x_ref, dst_ref=comm_ref.at[0],
            send_sem=send_sems.at[0], recv_sem=recv_sems.at[0],
            device_id=(left,), device_id_type=pl.DeviceIdType.MESH,
        )
        r1r = pltpu.make_async_remote_copy(
            src_ref=x_ref, dst_ref=comm_ref.at[1],
            send_sem=send_sems.at[1], recv_sem=recv_sems.at[1],
            device_id=(right,), device_id_type=pl.DeviceIdType.MESH,
        )
        r1l.start()
        r1r.start()

        gemm(x_ref[...], my)

        r1l.wait_recv()
        r2l = pltpu.make_async_remote_copy(
            src_ref=comm_ref.at[0, pl.ds(0, half), :],
            dst_ref=comm_ref.at[2, pl.ds(0, half), :],
            send_sem=send_sems.at[2], recv_sem=recv_sems.at[2],
            device_id=(left,), device_id_type=pl.DeviceIdType.MESH,
        )
        r2l.start()
        r1r.wait_recv()
        r2r = pltpu.make_async_remote_copy(
            src_ref=comm_ref.at[1, pl.ds(half, half), :],
            dst_ref=comm_ref.at[2, pl.ds(half, half), :],
            send_sem=send_sems.at[3], recv_sem=recv_sems.at[3],
            device_id=(right,), device_id_type=pl.DeviceIdType.MESH,
        )
        r2r.start()

        gemm(comm_ref[0], right)
        gemm(comm_ref[1], left)

        r2l.wait_recv()
        r2r.wait_recv()
        gemm(comm_ref[2], lax.rem(my + 2, N_DEV))

        r1l.wait_send()
        r1r.wait_send()
        r2l.wait_send()
        r2r.wait_send()

    return pl.pallas_call(
        body,
        out_shape=jax.ShapeDtypeStruct((N_DEV * m_per, n_per), jnp.float32),
        in_specs=[
            pl.BlockSpec(memory_space=pltpu.VMEM),
            pl.BlockSpec(memory_space=pltpu.VMEM),
            pl.BlockSpec(memory_space=pltpu.SMEM),
            pl.BlockSpec(memory_space=pltpu.SMEM),
        ],
        out_specs=pl.BlockSpec(memory_space=pltpu.VMEM),
        scratch_shapes=[
            pltpu.VMEM((3, m_per, k), jnp.float8_e5m2),
            pltpu.SemaphoreType.DMA((4,)),
            pltpu.SemaphoreType.DMA((4,)),
        ],
        compiler_params=pltpu.CompilerParams(
            collective_id=0,
            vmem_limit_bytes=100 * 1024 * 1024,
        ),
    )(x, w_mat, scale_x, scale_w)


# device time: 120222 ns/iter; 1.7788x vs baseline; 1.2005x over previous
import jax
import jax.numpy as jnp
from jax import lax
from jax.experimental import pallas as pl
from jax.experimental.pallas import tpu as pltpu

N_DEV = 4
W_CHUNKS = 4


def kernel(x, w_mat, scale_x, scale_w):
    m_per, k = x.shape
    _, n_per = w_mat.shape
    half = m_per // 2
    kc = k // W_CHUNKS

    x = x.astype(jnp.float8_e5m2)

    def body(x_ref, w_hbm, sx_ref, sw_ref, out_hbm,
             comm_ref, w8_ref, wstage, outbuf,
             send_sems, recv_sems, wdma_sems, odma_sems):
        my = lax.axis_index("i")
        left = lax.rem(my + (N_DEV - 1), N_DEV)
        right = lax.rem(my + 1, N_DEV)

        barrier_sem = pltpu.get_barrier_semaphore()
        for nbr in (left, right):
            pl.semaphore_signal(
                barrier_sem, inc=1,
                device_id=(nbr,), device_id_type=pl.DeviceIdType.MESH,
            )
        pl.semaphore_wait(barrier_sem, 2)

        r1l = pltpu.make_async_remote_copy(
            src_ref=x_ref, dst_ref=comm_ref.at[0],
            send_sem=send_sems.at[0], recv_sem=recv_sems.at[0],
            device_id=(left,), device_id_type=pl.DeviceIdType.MESH,
        )
        r1r = pltpu.make_async_remote_copy(
            src_ref=x_ref, dst_ref=comm_ref.at[1],
            send_sem=send_sems.at[1], recv_sem=recv_sems.at[1],
            device_id=(right,), device_id_type=pl.DeviceIdType.MESH,
        )
        r1l.start()
        r1r.start()

        wdma = []
        for i in range(W_CHUNKS):
            c = pltpu.make_async_copy(
                w_hbm.at[pl.ds(i * kc, kc), :],
                wstage.at[i % 2],
                wdma_sems.at[i % 2],
            )
            wdma.append(c)
        wdma[0].start()
        wdma[1].start()
        for i in range(W_CHUNKS):
            wdma[i].wait()
            if i + 2 < W_CHUNKS:
                wdma[i + 2].start()
            w8_ref[pl.ds(i * kc, kc), :] = wstage[i % 2].astype(jnp.float8_e5m2)

        scale = sx_ref[0] * sw_ref[0]
        odma = [None, None]

        def gemm_out(chunk, origin, slot):
            if odma[slot] is not None:
                odma[slot].wait()
            outbuf[slot] = (
                jnp.dot(chunk, w8_ref[...], preferred_element_type=jnp.float32)
                * scale
            )
            c = pltpu.make_async_copy(
                outbuf.at[slot],
                out_hbm.at[pl.ds(origin * m_per, m_per), :],
                odma_sems.at[slot],
            )
            c.start()
            odma[slot] = c

        gemm_out(x_ref[...], my, 0)

        r1l.wait_recv()
        r2l = pltpu.make_async_remote_copy(
            src_ref=comm_ref.at[0, pl.ds(0, half), :],
            dst_ref=comm_ref.at[2, pl.ds(0, half), :],
            send_sem=send_sems.at[2], recv_sem=recv_sems.at[2],
            device_id=(left,), device_id_type=pl.DeviceIdType.MESH,
        )
        r2l.start()
        r1r.wait_recv()
        r2r = pltpu.make_async_remote_copy(
            src_ref=comm_ref.at[1, pl.ds(half, half), :],
            dst_ref=comm_ref.at[2, pl.ds(half, half), :],
            send_sem=send_sems.at[3], recv_sem=recv_sems.at[3],
            device_id=(right,), device_id_type=pl.DeviceIdType.MESH,
        )
        r2r.start()

        gemm_out(comm_ref[0], right, 1)
        gemm_out(comm_ref[1], left, 0)

        r2l.wait_recv()
        r2r.wait_recv()
        gemm_out(comm_ref[2], lax.rem(my + 2, N_DEV), 1)

        odma[0].wait()
        odma[1].wait()
        r1l.wait_send()
        r1r.wait_send()
        r2l.wait_send()
        r2r.wait_send()

    return pl.pallas_call(
        body,
        out_shape=jax.ShapeDtypeStruct((N_DEV * m_per, n_per), jnp.float32),
        in_specs=[
            pl.BlockSpec(memory_space=pltpu.VMEM),
            pl.BlockSpec(memory_space=pl.ANY),
            pl.BlockSpec(memory_space=pltpu.SMEM),
            pl.BlockSpec(memory_space=pltpu.SMEM),
        ],
        out_specs=pl.BlockSpec(memory_space=pl.ANY),
        scratch_shapes=[
            pltpu.VMEM((3, m_per, k), jnp.float8_e5m2),
            pltpu.VMEM((k, n_per), jnp.float8_e5m2),
            pltpu.VMEM((2, k // W_CHUNKS, n_per), jnp.float32),
            pltpu.VMEM((2, m_per, n_per), jnp.float32),
            pltpu.SemaphoreType.DMA((4,)),
            pltpu.SemaphoreType.DMA((4,)),
            pltpu.SemaphoreType.DMA((2,)),
            pltpu.SemaphoreType.DMA((2,)),
        ],
        compiler_params=pltpu.CompilerParams(
            collective_id=0,
            vmem_limit_bytes=100 * 1024 * 1024,
        ),
    )(x, w_mat, scale_x, scale_w)


# device time: 112565 ns/iter; 1.8998x vs baseline; 1.0680x over previous
import jax
import jax.numpy as jnp
from jax import lax
from jax.experimental import pallas as pl
from jax.experimental.pallas import tpu as pltpu

N_DEV = 4
W_CHUNKS = 4


def kernel(x, w_mat, scale_x, scale_w):
    m_per, k = x.shape
    _, n_per = w_mat.shape
    half = m_per // 2
    quarter = m_per // 4
    kc = k // W_CHUNKS

    x = x.astype(jnp.float8_e5m2)

    def body(x_ref, w_hbm, sx_ref, sw_ref, out_hbm,
             comm_ref, w8_ref, wstage, obuf_l, obuf_h,
             send_sems, recv_sems, wdma_sems, odma_sems):
        my = lax.axis_index("i")
        left = lax.rem(my + (N_DEV - 1), N_DEV)
        right = lax.rem(my + 1, N_DEV)
        opp = lax.rem(my + 2, N_DEV)

        barrier_sem = pltpu.get_barrier_semaphore()
        for nbr in (left, right):
            pl.semaphore_signal(
                barrier_sem, inc=1,
                device_id=(nbr,), device_id_type=pl.DeviceIdType.MESH,
            )
        pl.semaphore_wait(barrier_sem, 2)

        def rdma(i, src, dst, rows, dev):
            return pltpu.make_async_remote_copy(
                src_ref=comm_ref.at[src, pl.ds(rows[0], rows[1]), :]
                if src is not None else x_ref.at[pl.ds(rows[0], rows[1]), :],
                dst_ref=comm_ref.at[dst, pl.ds(rows[0], rows[1]), :],
                send_sem=send_sems.at[i], recv_sem=recv_sems.at[i],
                device_id=(dev,), device_id_type=pl.DeviceIdType.MESH,
            )

        r1l0 = rdma(0, None, 0, (0, half), left)
        r1l1 = rdma(1, None, 0, (half, half), left)
        r1r0 = rdma(2, None, 1, (half, half), right)
        r1r1 = rdma(3, None, 1, (0, half), right)
        r1l0.start()
        r1l1.start()
        r1r0.start()
        r1r1.start()

        scale = sx_ref[0] * sw_ref[0]

        wdma = [
            pltpu.make_async_copy(
                w_hbm.at[pl.ds(i * kc, kc), :],
                wstage.at[i % 2],
                wdma_sems.at[i % 2],
            )
            for i in range(W_CHUNKS)
        ]
        wdma[0].start()
        wdma[1].start()
        for i in range(W_CHUNKS):
            wdma[i].wait()
            w8_ref[pl.ds(i * kc, kc), :] = wstage[i % 2].astype(jnp.float8_e5m2)
            if i + 2 < W_CHUNKS:
                wdma[i + 2].start()
            p = jnp.dot(x_ref[:, pl.ds(i * kc, kc)],
                        w8_ref[pl.ds(i * kc, kc), :],
                        preferred_element_type=jnp.float32)
            if i == 0:
                obuf_l[...] = p
            elif i == W_CHUNKS - 1:
                obuf_l[...] = (obuf_l[...] + p) * scale
            else:
                obuf_l[...] = obuf_l[...] + p
        odma_local = pltpu.make_async_copy(
            obuf_l, out_hbm.at[pl.ds(my * m_per, m_per), :], odma_sems.at[2],
        )
        odma_local.start()

        r1l0.wait_recv()
        r2l0 = rdma(4, 0, 2, (0, quarter), left)
        r2l1 = rdma(5, 0, 2, (quarter, quarter), left)
        r2l0.start()
        r2l1.start()
        r1r0.wait_recv()
        r2r0 = rdma(6, 1, 2, (3 * quarter, quarter), right)
        r2r1 = rdma(7, 1, 2, (2 * quarter, quarter), right)
        r2r0.start()
        r2r1.start()

        odma = [None, None]

        def gemm_out(src_slot, row0, nrows, origin_row0, slot):
            if odma[slot] is not None:
                odma[slot].wait()
            obuf_h[slot, pl.ds(0, nrows), :] = (
                jnp.dot(comm_ref[src_slot, pl.ds(row0, nrows), :], w8_ref[...],
                        preferred_element_type=jnp.float32)
                * scale
            )
            c = pltpu.make_async_copy(
                obuf_h.at[slot, pl.ds(0, nrows), :],
                out_hbm.at[pl.ds(origin_row0 + row0, nrows), :],
                odma_sems.at[slot],
            )
            c.start()
            odma[slot] = c

        gemm_out(0, 0, half, right * m_per, 0)
        gemm_out(1, half, half, left * m_per, 1)
        r1l1.wait_recv()
        gemm_out(0, half, half, right * m_per, 0)
        r1r1.wait_recv()
        gemm_out(1, 0, half, left * m_per, 1)

        r2l0.wait_recv()
        gemm_out(2, 0, quarter, opp * m_per, 0)
        r2r0.wait_recv()
        gemm_out(2, 3 * quarter, quarter, opp * m_per, 1)
        r2l1.wait_recv()
        gemm_out(2, quarter, quarter, opp * m_per, 0)
        r2r1.wait_recv()
        gemm_out(2, 2 * quarter, quarter, opp * m_per, 1)

        odma_local.wait()
        odma[0].wait()
        odma[1].wait()
        for r in (r1l0, r1l1, r1r0, r1r1, r2l0, r2l1, r2r0, r2r1):
            r.wait_send()

    return pl.pallas_call(
        body,
        out_shape=jax.ShapeDtypeStruct((N_DEV * m_per, n_per), jnp.float32),
        in_specs=[
            pl.BlockSpec(memory_space=pltpu.VMEM),
            pl.BlockSpec(memory_space=pl.ANY),
            pl.BlockSpec(memory_space=pltpu.SMEM),
            pl.BlockSpec(memory_space=pltpu.SMEM),
        ],
        out_specs=pl.BlockSpec(memory_space=pl.ANY),
        scratch_shapes=[
            pltpu.VMEM((3, m_per, k), jnp.float8_e5m2),
            pltpu.VMEM((k, n_per), jnp.float8_e5m2),
            pltpu.VMEM((2, k // W_CHUNKS, n_per), jnp.float32),
            pltpu.VMEM((m_per, n_per), jnp.float32),
            pltpu.VMEM((2, m_per // 2, n_per), jnp.float32),
            pltpu.SemaphoreType.DMA((8,)),
            pltpu.SemaphoreType.DMA((8,)),
            pltpu.SemaphoreType.DMA((2,)),
            pltpu.SemaphoreType.DMA((3,)),
        ],
        compiler_params=pltpu.CompilerParams(
            collective_id=0,
            vmem_limit_bytes=100 * 1024 * 1024,
        ),
    )(x, w_mat, scale_x, scale_w)


# device time: 110100 ns/iter; 1.9424x vs baseline; 1.0224x over previous
import jax
import jax.numpy as jnp
from jax import lax
from jax.experimental import pallas as pl
from jax.experimental.pallas import tpu as pltpu

N_DEV = 4
W_CHUNKS = 4
R1P = 4
R2P = 4


def kernel(x, w_mat, scale_x, scale_w):
    m_per, k = x.shape
    _, n_per = w_mat.shape
    q = m_per // R1P
    e = m_per // 2 // R2P
    kc = k // W_CHUNKS

    x = x.astype(jnp.float8_e5m2)

    r1_rows_l = [(i * q, q) for i in range(R1P)]
    r1_rows_r = [((R1P - 1 - i) * q, q) for i in range(R1P)]
    r2_rows_l = [(j * e, e) for j in range(R2P)]
    r2_rows_r = [(m_per - (j + 1) * e, e) for j in range(R2P)]

    def body(x_ref, w_hbm, sx_ref, sw_ref, out_hbm,
             comm_ref, w8_ref, wstage, obuf_l, obuf_h,
             send_sems, recv_sems, wdma_sems, odma_sems):
        my = lax.axis_index("i")
        left = lax.rem(my + (N_DEV - 1), N_DEV)
        right = lax.rem(my + 1, N_DEV)
        opp = lax.rem(my + 2, N_DEV)

        barrier_sem = pltpu.get_barrier_semaphore()
        for nbr in (left, right):
            pl.semaphore_signal(
                barrier_sem, inc=1,
                device_id=(nbr,), device_id_type=pl.DeviceIdType.MESH,
            )
        pl.semaphore_wait(barrier_sem, 2)

        def rdma(i, src, dst, rows, dev):
            return pltpu.make_async_remote_copy(
                src_ref=comm_ref.at[src, pl.ds(rows[0], rows[1]), :]
                if src is not None else x_ref.at[pl.ds(rows[0], rows[1]), :],
                dst_ref=comm_ref.at[dst, pl.ds(rows[0], rows[1]), :],
                send_sem=send_sems.at[i], recv_sem=recv_sems.at[i],
                device_id=(dev,), device_id_type=pl.DeviceIdType.MESH,
            )

        r1l = [rdma(i, None, 0, r1_rows_l[i], left) for i in range(R1P)]
        r1r = [rdma(R1P + i, None, 1, r1_rows_r[i], right) for i in range(R1P)]
        for r in r1l + r1r:
            r.start()

        scale = sx_ref[0] * sw_ref[0]

        wdma = [
            pltpu.make_async_copy(
                w_hbm.at[pl.ds(i * kc, kc), :],
                wstage.at[i % 2],
                wdma_sems.at[i % 2],
            )
            for i in range(W_CHUNKS)
        ]
        wdma[0].start()
        wdma[1].start()
        for i in range(W_CHUNKS):
            wdma[i].wait()
            w8_ref[pl.ds(i * kc, kc), :] = wstage[i % 2].astype(jnp.float8_e5m2)
            if i + 2 < W_CHUNKS:
                wdma[i + 2].start()
            p = jnp.dot(x_ref[:, pl.ds(i * kc, kc)],
                        w8_ref[pl.ds(i * kc, kc), :],
                        preferred_element_type=jnp.float32)
            if i == 0:
                obuf_l[...] = p
            elif i == W_CHUNKS - 1:
                obuf_l[...] = (obuf_l[...] + p) * scale
            else:
                obuf_l[...] = obuf_l[...] + p
        odma_local = pltpu.make_async_copy(
            obuf_l, out_hbm.at[pl.ds(my * m_per, m_per), :], odma_sems.at[2],
        )
        odma_local.start()

        r1l[0].wait_recv()
        r1l[1].wait_recv()
        r2l = [rdma(2 * R1P + j, 0, 2, r2_rows_l[j], left) for j in range(R2P)]
        for r in r2l:
            r.start()
        r1r[0].wait_recv()
        r1r[1].wait_recv()
        r2r = [rdma(2 * R1P + R2P + j, 1, 2, r2_rows_r[j], right)
               for j in range(R2P)]
        for r in r2r:
            r.start()

        odma = [None, None]

        def gemm_out(src_slot, row0, nrows, origin_row0, slot):
            if odma[slot] is not None:
                odma[slot].wait()
            obuf_h[slot, pl.ds(0, nrows), :] = (
                jnp.dot(comm_ref[src_slot, pl.ds(row0, nrows), :], w8_ref[...],
                        preferred_element_type=jnp.float32)
                * scale
            )
            c = pltpu.make_async_copy(
                obuf_h.at[slot, pl.ds(0, nrows), :],
                out_hbm.at[pl.ds(origin_row0 + row0, nrows), :],
                odma_sems.at[slot],
            )
            c.start()
            odma[slot] = c

        gemm_out(0, *r1_rows_l[0], right * m_per, 0)
        gemm_out(1, *r1_rows_r[0], left * m_per, 1)
        gemm_out(0, *r1_rows_l[1], right * m_per, 0)
        gemm_out(1, *r1_rows_r[1], left * m_per, 1)
        for i in range(2, R1P):
            r1l[i].wait_recv()
            gemm_out(0, *r1_rows_l[i], right * m_per, 0)
            r1r[i].wait_recv()
            gemm_out(1, *r1_rows_r[i], left * m_per, 1)

        for j in range(R2P):
            r2l[j].wait_recv()
            gemm_out(2, *r2_rows_l[j], opp * m_per, 0)
            r2r[j].wait_recv()
            gemm_out(2, *r2_rows_r[j], opp * m_per, 1)

        odma_local.wait()
        odma[0].wait()
        odma[1].wait()
        for r in r1l + r1r + r2l + r2r:
            r.wait_send()

    n_sems = 2 * R1P + 2 * R2P
    return pl.pallas_call(
        body,
        out_shape=jax.ShapeDtypeStruct((N_DEV * m_per, n_per), jnp.float32),
        in_specs=[
            pl.BlockSpec(memory_space=pltpu.VMEM),
            pl.BlockSpec(memory_space=pl.ANY),
            pl.BlockSpec(memory_space=pltpu.SMEM),
            pl.BlockSpec(memory_space=pltpu.SMEM),
        ],
        out_specs=pl.BlockSpec(memory_space=pl.ANY),
        scratch_shapes=[
            pltpu.VMEM((3, m_per, k), jnp.float8_e5m2),
            pltpu.VMEM((k, n_per), jnp.float8_e5m2),
            pltpu.VMEM((2, k // W_CHUNKS, n_per), jnp.float32),
            pltpu.VMEM((m_per, n_per), jnp.float32),
            pltpu.VMEM((2, m_per // 2, n_per), jnp.float32),
            pltpu.SemaphoreType.DMA((n_sems,)),
            pltpu.SemaphoreType.DMA((n_sems,)),
            pltpu.SemaphoreType.DMA((2,)),
            pltpu.SemaphoreType.DMA((3,)),
        ],
        compiler_params=pltpu.CompilerParams(
            collective_id=0,
            vmem_limit_bytes=100 * 1024 * 1024,
        ),
    )(x, w_mat, scale_x, scale_w)
